# baseline (speedup 1.0000x reference)
import jax
import jax.numpy as jnp
from jax.experimental import pallas as pl
from jax.experimental.pallas import tpu as pltpu
from jax.experimental.pallas import tpu_sc as plsc

E = 8
HIDDEN = 1024
INTER = 4096
EPS = 1e-12
S, L = 32, 128
NB = 8
BB = INTER // NB
NI = 4
BI = INTER // NI
RING = 8


def _route_body(task_hbm, hash_hbm, sched_hbm, task_out_hbm,
                t_s, h_s, dest_s, rank_s, cnt_s,
                start_s, ft_s, run_s, sched_s, sem):
    @pl.when(jax.lax.axis_index("c") == 0)
    def _():
        pltpu.async_copy(task_hbm, t_s, sem).wait()
        pltpu.async_copy(hash_hbm, h_s, sem).wait()

        @pl.loop(0, E)
        def _(e):
            cnt_s[e] = 0
            ft_s[e] = 0

        @pl.loop(0, S)
        def _(r):
            d = h_s[t_s[r], 0]
            dest_s[r] = d
            c = cnt_s[d]
            rank_s[r] = c
            ft_s[d] = jnp.where(c == 0, r, ft_s[d])
            cnt_s[d] = c + 1

        run_s[0] = 0

        @pl.loop(0, E)
        def _(e):
            start_s[e] = run_s[0]
            run_s[0] = run_s[0] + cnt_s[e]

        @pl.loop(0, S)
        def _(r):
            sched_s[4, r] = 0
            sched_s[5, r] = 0

        @pl.loop(0, E)
        def _(e):
            sched_s[4, e] = start_s[e]
            sched_s[5, e] = cnt_s[e]

            @pl.when(cnt_s[e] > 0)
            def _():
                p = start_s[e]
                sched_s[0, p] = e
                sched_s[1, p] = e
                sched_s[2, p] = 1
                sched_s[3, p] = ft_s[e]

        @pl.loop(0, S)
        def _(r):
            d = dest_s[r]

            @pl.when(rank_s[r] > 0)
            def _():
                p = start_s[d] + rank_s[r]
                sched_s[0, p] = E + r
                sched_s[1, p] = d
                sched_s[2, p] = 0
                sched_s[3, p] = r

        pltpu.async_copy(sched_s, sched_hbm, sem).wait()
        pltpu.async_copy(t_s, task_out_hbm, sem).wait()


def _routing_schedule(task_ids, hash_indices):
    route = pl.kernel(
        _route_body,
        out_type=[jax.ShapeDtypeStruct((6, S), jnp.int32),
                  jax.ShapeDtypeStruct((S,), jnp.int32)],
        mesh=plsc.ScalarSubcoreMesh(axis_name="c", num_cores=2),
        scratch_types=[pltpu.SMEM((S,), jnp.int32),
                       pltpu.SMEM((E, 1), jnp.int32),
                       pltpu.SMEM((S,), jnp.int32),
                       pltpu.SMEM((S,), jnp.int32),
                       pltpu.SMEM((E,), jnp.int32),
                       pltpu.SMEM((E,), jnp.int32),
                       pltpu.SMEM((E,), jnp.int32),
                       pltpu.SMEM((1,), jnp.int32),
                       pltpu.SMEM((6, S), jnp.int32),
                       pltpu.SemaphoreType.DMA],
    )
    return route(task_ids, hash_indices)


def _w1_copy(w1_hbm, ring1, sem1, g):
    e, b = divmod(g, NB)
    return pltpu.make_async_copy(
        w1_hbm.at[e, :, b * BB:(b + 1) * BB], ring1.at[g % RING],
        sem1.at[g % RING])


def _w2_copy(w2_hbm, ring2, sem2, g):
    e, b = divmod(g, NB)
    return pltpu.make_async_copy(
        w2_hbm.at[e, b * BB:(b + 1) * BB, :], ring2.at[g % RING],
        sem2.at[g % RING])


def _xrow_copy(x_hbm, xstage, xsem, row, slot):
    return pltpu.make_async_copy(x_hbm.at[row], xstage.at[slot],
                                 xsem.at[slot])


def _ffn_body(sched, b1_ref, b2_ref, lnw_ref, lnb_ref,
              x_hbm, w1_hbm, w2_hbm, o_hbm,
              fsum, w1c, w2c, xbc, ring1, ring2, xstage, xsum, ostage,
              sem1, sem2, xsem, osem):
    for g in range(RING):
        _w1_copy(w1_hbm, ring1, sem1, g).start()
        _w2_copy(w2_hbm, ring2, sem2, g).start()

    def item_loop(e):
        start = sched[4, e]
        cnt = sched[5, e]

        @pl.when(cnt > 0)
        def _():
            _xrow_copy(x_hbm, xstage, xsem, sched[3, start], 0).start()

            def srow(t, carry):
                slot = jax.lax.rem(t, 2)
                _xrow_copy(x_hbm, xstage, xsem, sched[3, start + t],
                           slot).wait()

                @pl.when(t + 1 < cnt)
                def _():
                    _xrow_copy(x_hbm, xstage, xsem, sched[3, start + t + 1],
                               jax.lax.rem(t + 1, 2)).start()

                @pl.when(t == 0)
                def _():
                    xsum[...] = xstage[slot]

                @pl.when(t > 0)
                def _():
                    xsum[...] += xstage[slot]

                return carry

            jax.lax.fori_loop(0, cnt, srow, 0)

        def item(k, carry):
            p = start + k
            is_sum = sched[2, p] == 1
            slot_in = jax.lax.rem(k, 2)

            @pl.when(k > 0)
            def _():
                _xrow_copy(x_hbm, xstage, xsem, sched[3, p], slot_in).wait()

            @pl.when(k + 1 < cnt)
            def _():
                _xrow_copy(x_hbm, xstage, xsem, sched[3, p + 1],
                           jax.lax.rem(k + 1, 2)).start()

            xv = jnp.where(is_sum, xsum[...], xstage[slot_in])
            xbc[...] = xv.astype(jnp.bfloat16)
            acc = None
            for jj in range(NI):
                y = jnp.dot(xbc[...], w1c[jj],
                            preferred_element_type=jnp.float32)
                y = y + b1_ref[e, 0, jj * BI:(jj + 1) * BI]
                y = y * 0.5 * (1.0 + jax.lax.erf(y * (2.0 ** -0.5)))
                pk = jnp.dot(y.astype(jnp.bfloat16), w2c[jj],
                             preferred_element_type=jnp.float32)
                acc = pk if acc is None else acc + pk
            zv = acc + b2_ref[e, 0] + xv
            mu = jnp.mean(zv, axis=-1, keepdims=True)
            d = zv - mu
            var = jnp.mean(d * d, axis=-1, keepdims=True)
            ln = d * jax.lax.rsqrt(var + EPS) * lnw_ref[e, 0] + lnb_ref[e, 0]
            prev_fs = fsum[...]
            fsum[...] = jnp.where(is_sum, ln, prev_fs)
            outv = ln + jnp.where(is_sum, jnp.zeros_like(ln), prev_fs)
            slot = jax.lax.rem(p, 2)
            orow = sched[3, p]

            @pl.when(p >= 2)
            def _():
                pltpu.make_async_copy(ostage.at[slot], o_hbm.at[orow],
                                      osem.at[slot]).wait()

            ostage[slot] = outv
            pltpu.make_async_copy(ostage.at[slot], o_hbm.at[orow],
                                  osem.at[slot]).start()
            return carry

        jax.lax.fori_loop(0, cnt, item, 0)

    for e in range(E):
        for b in range(NB):
            g = e * NB + b
            _w1_copy(w1_hbm, ring1, sem1, g).wait()
            _w2_copy(w2_hbm, ring2, sem2, g).wait()
            jj, half = divmod(b, NB // NI)
            w1c[jj, :, half * BB:(half + 1) * BB] = \
                ring1[g % RING].astype(jnp.bfloat16)
            w2c[jj, half * BB:(half + 1) * BB, :] = \
                ring2[g % RING].astype(jnp.bfloat16)
            if g + RING < E * NB:
                _w1_copy(w1_hbm, ring1, sem1, g + RING).start()
                _w2_copy(w2_hbm, ring2, sem2, g + RING).start()
        item_loop(e)

    pltpu.make_async_copy(ostage.at[0], o_hbm.at[0], osem.at[0]).wait()
    pltpu.make_async_copy(ostage.at[1], o_hbm.at[0], osem.at[1]).wait()


def _ffn(sched, x, W1, W2, b1r, b2r, lnwr, lnbr):
    grid_spec = pltpu.PrefetchScalarGridSpec(
        num_scalar_prefetch=1,
        grid=(1,),
        in_specs=[
            pl.BlockSpec((E, 1, INTER), lambda i, s: (0, 0, 0)),
            pl.BlockSpec((E, 1, HIDDEN), lambda i, s: (0, 0, 0)),
            pl.BlockSpec((E, 1, HIDDEN), lambda i, s: (0, 0, 0)),
            pl.BlockSpec((E, 1, HIDDEN), lambda i, s: (0, 0, 0)),
            pl.BlockSpec(memory_space=pl.ANY),
            pl.BlockSpec(memory_space=pl.ANY),
            pl.BlockSpec(memory_space=pl.ANY),
        ],
        out_specs=pl.BlockSpec(memory_space=pl.ANY),
        scratch_shapes=[pltpu.VMEM((L, HIDDEN), jnp.float32),
                        pltpu.VMEM((NI, HIDDEN, BI), jnp.bfloat16),
                        pltpu.VMEM((NI, BI, HIDDEN), jnp.bfloat16),
                        pltpu.VMEM((L, HIDDEN), jnp.bfloat16),
                        pltpu.VMEM((RING, HIDDEN, BB), jnp.float32),
                        pltpu.VMEM((RING, BB, HIDDEN), jnp.float32),
                        pltpu.VMEM((2, L, HIDDEN), jnp.float32),
                        pltpu.VMEM((L, HIDDEN), jnp.float32),
                        pltpu.VMEM((2, L, HIDDEN), jnp.float32),
                        pltpu.SemaphoreType.DMA((RING,)),
                        pltpu.SemaphoreType.DMA((RING,)),
                        pltpu.SemaphoreType.DMA((2,)),
                        pltpu.SemaphoreType.DMA((2,))],
    )
    return pl.pallas_call(
        _ffn_body,
        grid_spec=grid_spec,
        out_shape=jax.ShapeDtypeStruct((S, L, HIDDEN), jnp.float32),
    )(sched, b1r, b2r, lnwr, lnbr, x, W1, W2)


def kernel(x, task_ids, hash_indices, W1, b1, W2, b2, ln_w, ln_b):
    sched, task_sc = _routing_schedule(task_ids, hash_indices)
    out = _ffn(sched, x, W1, W2,
               b1.reshape(E, 1, INTER), b2.reshape(E, 1, HIDDEN),
               ln_w.reshape(E, 1, HIDDEN), ln_b.reshape(E, 1, HIDDEN))
    return (out, task_sc)


# speedup vs baseline: 1.0488x; 1.0488x over previous
import jax
import jax.numpy as jnp
from jax.experimental import pallas as pl
from jax.experimental.pallas import tpu as pltpu
from jax.experimental.pallas import tpu_sc as plsc

E = 8
HIDDEN = 1024
INTER = 4096
EPS = 1e-12
S, L = 32, 128
NB = 8
BB = INTER // NB
NI = 4
BI = INTER // NI
RING = 5


def _route_body(task_hbm, hash_hbm, sched_hbm, dm_hbm, task_out_hbm,
                t_s, h_s, dest_s, rank_s, cnt_s,
                start_s, ft_s, run_s, sched_s, dm_s, sem):
    @pl.when(jax.lax.axis_index("c") == 0)
    def _():
        pltpu.async_copy(task_hbm, t_s, sem).wait()
        pltpu.async_copy(hash_hbm, h_s, sem).wait()

        @pl.loop(0, E)
        def _(e):
            cnt_s[e] = 0
            ft_s[e] = 0

        @pl.loop(0, S)
        def _(r):
            d = h_s[t_s[r], 0]
            dest_s[r] = d
            c = cnt_s[d]
            rank_s[r] = c
            ft_s[d] = jnp.where(c == 0, r, ft_s[d])
            cnt_s[d] = c + 1

            @pl.loop(0, S)
            def _(q):
                dm_s[r, q] = 0.0

        run_s[0] = 0

        @pl.loop(0, E)
        def _(e):
            start_s[e] = run_s[0]
            run_s[0] = run_s[0] + cnt_s[e]

        @pl.loop(0, S)
        def _(r):
            sched_s[4, r] = 0
            sched_s[5, r] = 0

        @pl.loop(0, E)
        def _(e):
            sched_s[4, e] = start_s[e]
            sched_s[5, e] = cnt_s[e]

            @pl.when(cnt_s[e] > 0)
            def _():
                p = start_s[e]
                sched_s[0, p] = e
                sched_s[1, p] = e
                sched_s[2, p] = 1
                sched_s[3, p] = ft_s[e]

        @pl.loop(0, S)
        def _(r):
            d = dest_s[r]
            dm_s[start_s[d], r] = 1.0

            @pl.when(rank_s[r] > 0)
            def _():
                p = start_s[d] + rank_s[r]
                sched_s[0, p] = E + r
                sched_s[1, p] = d
                sched_s[2, p] = 0
                sched_s[3, p] = r
                dm_s[p, r] = 1.0

        pltpu.async_copy(sched_s, sched_hbm, sem).wait()
        pltpu.async_copy(dm_s, dm_hbm, sem).wait()
        pltpu.async_copy(t_s, task_out_hbm, sem).wait()


def _routing_schedule(task_ids, hash_indices):
    route = pl.kernel(
        _route_body,
        out_type=[jax.ShapeDtypeStruct((6, S), jnp.int32),
                  jax.ShapeDtypeStruct((S, S), jnp.float32),
                  jax.ShapeDtypeStruct((S,), jnp.int32)],
        mesh=plsc.ScalarSubcoreMesh(axis_name="c", num_cores=2),
        scratch_types=[pltpu.SMEM((S,), jnp.int32),
                       pltpu.SMEM((E, 1), jnp.int32),
                       pltpu.SMEM((S,), jnp.int32),
                       pltpu.SMEM((S,), jnp.int32),
                       pltpu.SMEM((E,), jnp.int32),
                       pltpu.SMEM((E,), jnp.int32),
                       pltpu.SMEM((E,), jnp.int32),
                       pltpu.SMEM((1,), jnp.int32),
                       pltpu.SMEM((6, S), jnp.int32),
                       pltpu.SMEM((S, S), jnp.float32),
                       pltpu.SemaphoreType.DMA],
    )
    return route(task_ids, hash_indices)


def _gather_body(dm_ref, x_ref, t_ref, o_ref, to_ref):
    o_ref[...] = jax.lax.dot(dm_ref[...], x_ref[...],
                             precision=jax.lax.Precision.HIGHEST,
                             preferred_element_type=jnp.float32)
    to_ref[...] = t_ref[...]


def _dispatch_gather(dm, xf, task_ids):
    CH = 16384
    return pl.pallas_call(
        _gather_body,
        grid=(xf.shape[1] // CH,),
        in_specs=[pl.BlockSpec((S, S), lambda c: (0, 0)),
                  pl.BlockSpec((S, CH), lambda c: (0, c)),
                  pl.BlockSpec((1, S), lambda c: (0, 0))],
        out_specs=[pl.BlockSpec((S, CH), lambda c: (0, c)),
                   pl.BlockSpec((1, S), lambda c: (0, 0))],
        out_shape=[jax.ShapeDtypeStruct((S, xf.shape[1]), jnp.float32),
                   jax.ShapeDtypeStruct((1, S), jnp.int32)],
    )(dm, xf, task_ids.reshape(1, S))


def _w1_copy(w1_hbm, ring1, sem1, g):
    e, b = divmod(g, NB)
    return pltpu.make_async_copy(
        w1_hbm.at[e, :, b * BB:(b + 1) * BB], ring1.at[g % RING],
        sem1.at[g % RING])


def _w2_copy(w2_hbm, ring2, sem2, g):
    e, b = divmod(g, NB)
    return pltpu.make_async_copy(
        w2_hbm.at[e, b * BB:(b + 1) * BB, :], ring2.at[g % RING],
        sem2.at[g % RING])


def _ffn_body(sched, x_ref, b1_ref, b2_ref, lnw_ref, lnb_ref,
              w1_hbm, w2_hbm, o_hbm,
              fsum, w1c, w2c, xbc, ring1, ring2, ostage,
              sem1, sem2, osem):
    for g in range(RING):
        _w1_copy(w1_hbm, ring1, sem1, g).start()
        _w2_copy(w2_hbm, ring2, sem2, g).start()

    def item_loop(e):
        start = sched[4, e]
        cnt = sched[5, e]

        def item(k, carry):
            p = start + k
            is_sum = sched[2, p] == 1
            xbc[...] = x_ref[p].astype(jnp.bfloat16)
            acc = None
            for jj in range(NI):
                y = jnp.dot(xbc[...], w1c[jj],
                            preferred_element_type=jnp.float32)
                y = y + b1_ref[e, 0, jj * BI:(jj + 1) * BI]
                y = y * 0.5 * (1.0 + jax.lax.erf(y * (2.0 ** -0.5)))
                pk = jnp.dot(y.astype(jnp.bfloat16), w2c[jj],
                             preferred_element_type=jnp.float32)
                acc = pk if acc is None else acc + pk
            zv = acc + b2_ref[e, 0] + x_ref[p]
            mu = jnp.mean(zv, axis=-1, keepdims=True)
            d = zv - mu
            var = jnp.mean(d * d, axis=-1, keepdims=True)
            ln = d * jax.lax.rsqrt(var + EPS) * lnw_ref[e, 0] + lnb_ref[e, 0]
            prev_fs = fsum[...]
            fsum[...] = jnp.where(is_sum, ln, prev_fs)
            outv = ln + jnp.where(is_sum, jnp.zeros_like(ln), prev_fs)
            slot = jax.lax.rem(p, 2)
            orow = sched[3, p]

            @pl.when(p >= 2)
            def _():
                pltpu.make_async_copy(ostage.at[slot], o_hbm.at[orow],
                                      osem.at[slot]).wait()

            ostage[slot] = outv
            pltpu.make_async_copy(ostage.at[slot], o_hbm.at[orow],
                                  osem.at[slot]).start()
            return carry

        jax.lax.fori_loop(0, cnt, item, 0)

    for e in range(E):
        for b in range(NB):
            g = e * NB + b
            _w1_copy(w1_hbm, ring1, sem1, g).wait()
            _w2_copy(w2_hbm, ring2, sem2, g).wait()
            jj, half = divmod(b, NB // NI)
            w1c[jj, :, half * BB:(half + 1) * BB] = \
                ring1[g % RING].astype(jnp.bfloat16)
            w2c[jj, half * BB:(half + 1) * BB, :] = \
                ring2[g % RING].astype(jnp.bfloat16)
            if g + RING < E * NB:
                _w1_copy(w1_hbm, ring1, sem1, g + RING).start()
                _w2_copy(w2_hbm, ring2, sem2, g + RING).start()
        item_loop(e)

    pltpu.make_async_copy(ostage.at[0], o_hbm.at[0], osem.at[0]).wait()
    pltpu.make_async_copy(ostage.at[1], o_hbm.at[0], osem.at[1]).wait()


def _ffn(sched, xs, W1, W2, b1r, b2r, lnwr, lnbr):
    grid_spec = pltpu.PrefetchScalarGridSpec(
        num_scalar_prefetch=1,
        grid=(1,),
        in_specs=[
            pl.BlockSpec((S, L, HIDDEN), lambda i, s: (0, 0, 0)),
            pl.BlockSpec((E, 1, INTER), lambda i, s: (0, 0, 0)),
            pl.BlockSpec((E, 1, HIDDEN), lambda i, s: (0, 0, 0)),
            pl.BlockSpec((E, 1, HIDDEN), lambda i, s: (0, 0, 0)),
            pl.BlockSpec((E, 1, HIDDEN), lambda i, s: (0, 0, 0)),
            pl.BlockSpec(memory_space=pl.ANY),
            pl.BlockSpec(memory_space=pl.ANY),
        ],
        out_specs=pl.BlockSpec(memory_space=pl.ANY),
        scratch_shapes=[pltpu.VMEM((L, HIDDEN), jnp.float32),
                        pltpu.VMEM((NI, HIDDEN, BI), jnp.bfloat16),
                        pltpu.VMEM((NI, BI, HIDDEN), jnp.bfloat16),
                        pltpu.VMEM((L, HIDDEN), jnp.bfloat16),
                        pltpu.VMEM((RING, HIDDEN, BB), jnp.float32),
                        pltpu.VMEM((RING, BB, HIDDEN), jnp.float32),
                        pltpu.VMEM((2, L, HIDDEN), jnp.float32),
                        pltpu.SemaphoreType.DMA((RING,)),
                        pltpu.SemaphoreType.DMA((RING,)),
                        pltpu.SemaphoreType.DMA((2,))],
    )
    return pl.pallas_call(
        _ffn_body,
        grid_spec=grid_spec,
        out_shape=jax.ShapeDtypeStruct((S, L, HIDDEN), jnp.float32),
    )(sched, xs, b1r, b2r, lnwr, lnbr, W1, W2)


def kernel(x, task_ids, hash_indices, W1, b1, W2, b2, ln_w, ln_b):
    sched, dm, _ = _routing_schedule(task_ids, hash_indices)
    xsf, task_out = _dispatch_gather(dm, x.reshape(S, L * HIDDEN), task_ids)
    out = _ffn(sched, xsf.reshape(S, L, HIDDEN), W1, W2,
               b1.reshape(E, 1, INTER), b2.reshape(E, 1, HIDDEN),
               ln_w.reshape(E, 1, HIDDEN), ln_b.reshape(E, 1, HIDDEN))
    return (out, task_out.reshape(S))
